# baseline (speedup 1.0000x reference)
import functools

import jax
import jax.numpy as jnp
from jax import lax
from jax.experimental import pallas as pl
from jax.experimental.pallas import tpu as pltpu
from jax.experimental.pallas import tpu_sc as plsc

B = 16384
D = 64
NC = 2
NS = 16
NW = NC * NS
BPW = B // NW
C = 128


def _tec_body(head_hbm, rel_hbm, tail_hbm, ent_hbm, relt_hbm, nrm_hbm,
              out_hbm, hidx, tidx, ridx, hoff, toff, roff,
              hrows, trows, rrows, wrows, oscr, sem):
    wid = lax.axis_index("s") * NC + lax.axis_index("c")
    base = wid * BPW
    ent2, relt2, nrm2 = ent_hbm, relt_hbm, nrm_hbm

    pltpu.sync_copy(head_hbm.at[pl.ds(base, BPW)], hidx)
    pltpu.sync_copy(tail_hbm.at[pl.ds(base, BPW)], tidx)
    pltpu.sync_copy(rel_hbm.at[pl.ds(base, BPW)], ridx)

    def split(g, _):
        hv = hidx[pl.ds(g * 16, 16)]
        tv = tidx[pl.ds(g * 16, 16)]
        rv = ridx[pl.ds(g * 16, 16)]
        hoff[pl.ds(g * 16, 16)] = (hv & 1) << 6
        toff[pl.ds(g * 16, 16)] = (tv & 1) << 6
        roff[pl.ds(g * 16, 16)] = (rv & 1) << 6
        hidx[pl.ds(g * 16, 16)] = hv >> 1
        tidx[pl.ds(g * 16, 16)] = tv >> 1
        ridx[pl.ds(g * 16, 16)] = rv >> 1
        return _

    lax.fori_loop(0, BPW // 16, split, None)

    lane = lax.iota(jnp.int32, 16)

    for c in range(BPW // C):
        off = c * C
        cph = pltpu.async_copy(ent2.at[hidx.at[pl.ds(off, C)]], hrows, sem)
        cpt = pltpu.async_copy(ent2.at[tidx.at[pl.ds(off, C)]], trows, sem)
        cpr = pltpu.async_copy(relt2.at[ridx.at[pl.ds(off, C)]], rrows, sem)
        cpw = pltpu.async_copy(nrm2.at[ridx.at[pl.ds(off, C)]], wrows, sem)
        cph.wait()
        cpt.wait()
        cpr.wait()
        cpw.wait()

        def group(g, carry):
            acc = jnp.zeros((16,), jnp.float32)
            phv = hoff[pl.ds(off + g * 16, 16)]
            ptv = toff[pl.ds(off + g * 16, 16)]
            prv = roff[pl.ds(off + g * 16, 16)]
            for j in range(16):
                e = g * 16 + j
                ph = phv[j]
                pt = ptv[j]
                pr = prv[j]
                u0 = hrows[e, pl.ds(ph, 16)] - trows[e, pl.ds(pt, 16)]
                u1 = hrows[e, pl.ds(ph + 16, 16)] - trows[e, pl.ds(pt + 16, 16)]
                u2 = hrows[e, pl.ds(ph + 32, 16)] - trows[e, pl.ds(pt + 32, 16)]
                u3 = hrows[e, pl.ds(ph + 48, 16)] - trows[e, pl.ds(pt + 48, 16)]
                w0 = wrows[e, pl.ds(pr, 16)]
                w1 = wrows[e, pl.ds(pr + 16, 16)]
                w2 = wrows[e, pl.ds(pr + 32, 16)]
                w3 = wrows[e, pl.ds(pr + 48, 16)]
                m = (u0 * w0 + u1 * w1) + (u2 * w2 + u3 * w3)
                a = jnp.sum(m)
                x0 = u0 + rrows[e, pl.ds(pr, 16)] - a * w0
                x1 = u1 + rrows[e, pl.ds(pr + 16, 16)] - a * w1
                x2 = u2 + rrows[e, pl.ds(pr + 32, 16)] - a * w2
                x3 = u3 + rrows[e, pl.ds(pr + 48, 16)] - a * w3
                s = (jnp.abs(x0) + jnp.abs(x1)) + (jnp.abs(x2) + jnp.abs(x3))
                acc = jnp.where(lane == j, jnp.sum(s), acc)
            oscr[pl.ds(off + g * 16, 16)] = acc
            return carry

        lax.fori_loop(0, C // 16, group, None)

    pltpu.sync_copy(oscr, out_hbm.at[pl.ds(base, BPW)])


def kernel(head, relation, tail, entity_table, relation_table, normal_table):
    mesh = plsc.VectorSubcoreMesh(core_axis_name="c", subcore_axis_name="s")
    k = functools.partial(
        pl.kernel,
        mesh=mesh,
        compiler_params=pltpu.CompilerParams(needs_layout_passes=False),
        out_type=jax.ShapeDtypeStruct((B,), jnp.float32),
        scratch_types=[
            pltpu.VMEM((BPW,), jnp.int32),
            pltpu.VMEM((BPW,), jnp.int32),
            pltpu.VMEM((BPW,), jnp.int32),
            pltpu.VMEM((BPW,), jnp.int32),
            pltpu.VMEM((BPW,), jnp.int32),
            pltpu.VMEM((BPW,), jnp.int32),
            pltpu.VMEM((C, 128), jnp.float32),
            pltpu.VMEM((C, 128), jnp.float32),
            pltpu.VMEM((C, 128), jnp.float32),
            pltpu.VMEM((C, 128), jnp.float32),
            pltpu.VMEM((BPW,), jnp.float32),
            pltpu.SemaphoreType.DMA,
        ],
    )(_tec_body)
    ent2 = jnp.reshape(entity_table, (500000, 128))
    relt2 = jnp.reshape(relation_table, (500, 128))
    nrm2 = jnp.reshape(normal_table, (500, 128))
    return k(head, relation, tail, ent2, relt2, nrm2)


# speedup vs baseline: 1.4990x; 1.4990x over previous
import functools

import jax
import jax.numpy as jnp
from jax import lax
from jax.experimental import pallas as pl
from jax.experimental.pallas import tpu as pltpu
from jax.experimental.pallas import tpu_sc as plsc

B = 16384
D = 64
NC = 2
NS = 16
NW = NC * NS
BPW = B // NW
C = 32


def _tec_body(head_hbm, rel_hbm, tail_hbm, ent_hbm, relt_hbm, nrm_hbm,
              out_hbm, hidx, tidx, ridx, roff, hstage, tstage, rrows,
              wrows, oscr, sem):
    wid = lax.axis_index("s") * NC + lax.axis_index("c")
    base = wid * BPW

    pltpu.sync_copy(head_hbm.at[pl.ds(base, BPW)], hidx)
    pltpu.sync_copy(tail_hbm.at[pl.ds(base, BPW)], tidx)
    pltpu.sync_copy(rel_hbm.at[pl.ds(base, BPW)], ridx)

    def split(g, _):
        rv = ridx[pl.ds(g * 16, 16)]
        roff[pl.ds(g * 16, 16)] = (rv & 1) << 6
        ridx[pl.ds(g * 16, 16)] = rv >> 1
        return _

    lax.fori_loop(0, BPW // 16, split, None)

    lane = lax.iota(jnp.int32, 16)

    def chunk(c, carry0):
        off = c * C

        def fire(g, _):
            hv = hidx[pl.ds(off + g * 16, 16)]
            tv = tidx[pl.ds(off + g * 16, 16)]
            for j in range(16):
                el = g * 16 + j
                hs = pl.multiple_of((hv[j] >> 3) * 8, 8)
                ts = pl.multiple_of((tv[j] >> 3) * 8, 8)
                pltpu.async_copy(ent_hbm.at[pl.ds(hs, 8), :],
                                 hstage.at[pl.ds(el * 8, 8), :], sem)
                pltpu.async_copy(ent_hbm.at[pl.ds(ts, 8), :],
                                 tstage.at[pl.ds(el * 8, 8), :], sem)
            return _

        lax.fori_loop(0, C // 16, fire, None)
        cpr = pltpu.async_copy(relt_hbm.at[ridx.at[pl.ds(off, C)]], rrows,
                               sem)
        cpw = pltpu.async_copy(nrm_hbm.at[ridx.at[pl.ds(off, C)]], wrows,
                               sem)
        pltpu.make_async_copy(ent_hbm.at[pl.ds(0, C * 8), :], hstage,
                              sem).wait()
        pltpu.make_async_copy(ent_hbm.at[pl.ds(0, C * 8), :], tstage,
                              sem).wait()
        cpr.wait()
        cpw.wait()

        def group(g, carry):
            acc = jnp.zeros((16,), jnp.float32)
            prv = roff[pl.ds(off + g * 16, 16)]
            hv = hidx[pl.ds(off + g * 16, 16)]
            tv = tidx[pl.ds(off + g * 16, 16)]
            for j in range(16):
                e = g * 16 + j
                pr = prv[j]
                hr = e * 8 + (hv[j] & 7)
                tr = e * 8 + (tv[j] & 7)
                u0 = hstage[hr, pl.ds(0, 16)] - tstage[tr, pl.ds(0, 16)]
                u1 = hstage[hr, pl.ds(16, 16)] - tstage[tr, pl.ds(16, 16)]
                u2 = hstage[hr, pl.ds(32, 16)] - tstage[tr, pl.ds(32, 16)]
                u3 = hstage[hr, pl.ds(48, 16)] - tstage[tr, pl.ds(48, 16)]
                w0 = wrows[e, pl.ds(pr, 16)]
                w1 = wrows[e, pl.ds(pr + 16, 16)]
                w2 = wrows[e, pl.ds(pr + 32, 16)]
                w3 = wrows[e, pl.ds(pr + 48, 16)]
                m = (u0 * w0 + u1 * w1) + (u2 * w2 + u3 * w3)
                a = jnp.sum(m)
                x0 = u0 + rrows[e, pl.ds(pr, 16)] - a * w0
                x1 = u1 + rrows[e, pl.ds(pr + 16, 16)] - a * w1
                x2 = u2 + rrows[e, pl.ds(pr + 32, 16)] - a * w2
                x3 = u3 + rrows[e, pl.ds(pr + 48, 16)] - a * w3
                s = (jnp.abs(x0) + jnp.abs(x1)) + (jnp.abs(x2) + jnp.abs(x3))
                acc = jnp.where(lane == j, jnp.sum(s), acc)
            oscr[pl.ds(off + g * 16, 16)] = acc
            return carry

        lax.fori_loop(0, C // 16, group, None)
        return carry0

    lax.fori_loop(0, BPW // C, chunk, None)

    pltpu.sync_copy(oscr, out_hbm.at[pl.ds(base, BPW)])


def kernel(head, relation, tail, entity_table, relation_table, normal_table):
    mesh = plsc.VectorSubcoreMesh(core_axis_name="c", subcore_axis_name="s")
    k = functools.partial(
        pl.kernel,
        mesh=mesh,
        compiler_params=pltpu.CompilerParams(needs_layout_passes=False),
        out_type=jax.ShapeDtypeStruct((B,), jnp.float32),
        scratch_types=[
            pltpu.VMEM((BPW,), jnp.int32),
            pltpu.VMEM((BPW,), jnp.int32),
            pltpu.VMEM((BPW,), jnp.int32),
            pltpu.VMEM((BPW,), jnp.int32),
            pltpu.VMEM((C * 8, D), jnp.float32),
            pltpu.VMEM((C * 8, D), jnp.float32),
            pltpu.VMEM((C, 128), jnp.float32),
            pltpu.VMEM((C, 128), jnp.float32),
            pltpu.VMEM((BPW,), jnp.float32),
            pltpu.SemaphoreType.DMA,
        ],
    )(_tec_body)
    relt2 = jnp.reshape(relation_table, (500, 128))
    nrm2 = jnp.reshape(normal_table, (500, 128))
    return k(head, relation, tail, entity_table, relt2, nrm2)
